# baseline (speedup 1.0000x reference)
import functools

import jax
import jax.numpy as jnp
from jax import lax
from jax.experimental import pallas as pl
from jax.experimental.pallas import tpu as pltpu
from jax.experimental.pallas import tpu_sc as plsc

N = 10000
E = 320000
D_IN = 128
HID = 64
D_OUT = 128

NC = 2
NS = 16
NWORK = NC * NS
WIN = 128
EPW = E // NWORK
NWIN_V = -(-EPW // WIN)
TAIL = EPW - (NWIN_V - 1) * WIN
KSLOT = 8
WPS = ((NWIN_V + KSLOT - 1) // KSLOT) * KSLOT
NBATCH = WPS // KSLOT
N_PAD = 10240
RPS = N_PAD // NS
DEGW = 16

_mesh = plsc.VectorSubcoreMesh(core_axis_name="c", subcore_axis_name="s")
_sc_params = pltpu.CompilerParams(use_tc_tiling_on_sc=False)


@functools.partial(
    pl.kernel,
    out_type=jax.ShapeDtypeStruct((NC, N_PAD, DEGW), jnp.float32),
    mesh=_mesh,
    scratch_types=[
        pltpu.VMEM((WPS, WIN), jnp.int32),
        pltpu.VMEM((WIN, DEGW), jnp.float32),
        pltpu.VMEM((WIN, DEGW), jnp.float32),
        pltpu.VMEM_SHARED((N_PAD, DEGW), jnp.float32),
    ],
    compiler_params=_sc_params,
)
def _deg_kernel(dstw_hbm, vfull_hbm, vtail_hbm, zeros_hbm, out_hbm,
                dst_v, vfull, vtail, acc):
    c = lax.axis_index("c")
    s = lax.axis_index("s")
    wid = c * NS + s
    r0 = s * RPS
    pltpu.sync_copy(zeros_hbm.at[pl.ds(r0, RPS)], acc.at[pl.ds(r0, RPS)])
    pltpu.sync_copy(dstw_hbm.at[wid], dst_v)
    pltpu.sync_copy(vfull_hbm, vfull)
    pltpu.sync_copy(vtail_hbm, vtail)
    plsc.subcore_barrier()

    @pl.loop(0, NWIN_V - 1)
    def _(w):
        pltpu.sync_copy(vfull, acc.at[dst_v.at[w]], add=True)

    pltpu.sync_copy(vtail, acc.at[dst_v.at[NWIN_V - 1]], add=True)
    plsc.subcore_barrier()
    pltpu.sync_copy(acc.at[pl.ds(r0, RPS)], out_hbm.at[c, pl.ds(r0, RPS)])


@functools.partial(
    pl.kernel,
    out_type=jax.ShapeDtypeStruct((NC, N_PAD, HID), jnp.float32),
    mesh=_mesh,
    scratch_types=[
        pltpu.VMEM((WPS, WIN), jnp.int32),
        pltpu.VMEM((WPS, WIN), jnp.int32),
    ] + [pltpu.VMEM((WIN, HID), jnp.float32) for _ in range(KSLOT)] + [
        pltpu.VMEM_SHARED((N_PAD, HID), jnp.float32),
    ] + [pltpu.SemaphoreType.DMA for _ in range(2 * KSLOT)],
    compiler_params=_sc_params,
)
def _msg_kernel(hs_hbm, srcw_hbm, dstw_hbm, zeros_hbm, out_hbm,
                src_v, dst_v, *rest):
    bufs = rest[:KSLOT]
    acc = rest[KSLOT]
    gsem = rest[KSLOT + 1:2 * KSLOT + 1]
    ssem = rest[2 * KSLOT + 1:]
    c = lax.axis_index("c")
    s = lax.axis_index("s")
    wid = c * NS + s
    r0 = s * RPS
    pltpu.sync_copy(zeros_hbm.at[pl.ds(r0, RPS)], acc.at[pl.ds(r0, RPS)])
    pltpu.sync_copy(srcw_hbm.at[wid], src_v)
    pltpu.sync_copy(dstw_hbm.at[wid], dst_v)
    plsc.subcore_barrier()

    def _wait_gather(i):
        pltpu.make_async_copy(hs_hbm.at[pl.ds(0, WIN)], bufs[i], gsem[i]).wait()

    def _wait_scatter(i):
        pltpu.make_async_copy(bufs[i], acc.at[pl.ds(0, WIN)], ssem[i]).wait()

    for i in range(KSLOT):
        pltpu.async_copy(hs_hbm.at[src_v.at[i]], bufs[i], gsem[i])

    @pl.loop(0, WPS - KSLOT, step=KSLOT)
    def _(w):
        for i in range(KSLOT):
            _wait_gather(i)
            pltpu.async_copy(bufs[i], acc.at[dst_v.at[w + i]], ssem[i],
                             add=True)
        for i in range(KSLOT):
            _wait_scatter(i)
            pltpu.async_copy(hs_hbm.at[src_v.at[w + KSLOT + i]], bufs[i],
                             gsem[i])

    for i in range(KSLOT):
        _wait_gather(i)
        pltpu.async_copy(bufs[i], acc.at[dst_v.at[WPS - KSLOT + i]], ssem[i],
                         add=True)
    for i in range(KSLOT):
        _wait_scatter(i)
    plsc.subcore_barrier()
    pltpu.sync_copy(acc.at[pl.ds(r0, RPS)], out_hbm.at[c, pl.ds(r0, RPS)])


def _t1_body(x_ref, w1_ref, degp_ref, hs_ref, dinvb_ref):
    h = jnp.dot(x_ref[...], w1_ref[...], preferred_element_type=jnp.float32)
    deg = degp_ref[0:N_PAD, 0:1] + degp_ref[N_PAD:2 * N_PAD, 0:1] + 1.0
    dinv = lax.rsqrt(deg)
    rows = lax.broadcasted_iota(jnp.int32, (N_PAD, 1), 0)
    dinv = jnp.where(rows < N, dinv, 0.0)
    dinvb = jnp.broadcast_to(dinv, (N_PAD, HID))
    dinvb_ref[...] = dinvb
    hs_ref[...] = h * dinvb


def _t2_body(agg_ref, hs_ref, dinvb_ref, b1_ref, h2s_ref):
    tot = agg_ref[0:N_PAD, :] + agg_ref[N_PAD:2 * N_PAD, :] + hs_ref[...]
    dinvb = dinvb_ref[...]
    out1 = dinvb * tot + b1_ref[...]
    r = jnp.maximum(out1, 0.0)
    ss = jnp.sum(r * r, axis=1, keepdims=True)
    nrm = jnp.maximum(jnp.sqrt(ss), 1e-12)
    h2s_ref[...] = (r / nrm) * dinvb


def _t3_body(agg_ref, h2s_ref, dinvb_ref, w2_ref, b2_ref, out_ref):
    pre = dinvb_ref[...] * (
        agg_ref[0:N_PAD, :] + agg_ref[N_PAD:2 * N_PAD, :] + h2s_ref[...])
    out_ref[...] = (
        jnp.dot(pre, w2_ref[...], preferred_element_type=jnp.float32)
        + b2_ref[...])


_f32 = jnp.float32


def kernel(x, edge_index, W1, b1, W2, b2):
    src = edge_index[0].reshape(NWORK, EPW)
    dst = edge_index[1].reshape(NWORK, EPW)
    pad_n = WPS * WIN - EPW
    srcw = jnp.concatenate(
        [src, jnp.full((NWORK, pad_n), N, jnp.int32)], axis=1
    ).reshape(NWORK, WPS, WIN)
    dstw = jnp.concatenate(
        [dst, jnp.zeros((NWORK, pad_n), jnp.int32)], axis=1
    ).reshape(NWORK, WPS, WIN)

    x_pad = jnp.zeros((N_PAD, D_IN), _f32).at[:N].set(x)
    zeros_deg = jnp.zeros((N_PAD, DEGW), _f32)
    zeros_hid = jnp.zeros((N_PAD, HID), _f32)
    vfull = jnp.ones((WIN, DEGW), _f32)
    vtail = jnp.zeros((WIN, DEGW), _f32).at[:TAIL].set(1.0)

    degp = _deg_kernel(dstw, vfull, vtail, zeros_deg)
    degp2 = degp.reshape(NC * N_PAD, DEGW)

    hs, dinvb = pl.pallas_call(
        _t1_body,
        out_shape=(jax.ShapeDtypeStruct((N_PAD, HID), _f32),
                   jax.ShapeDtypeStruct((N_PAD, HID), _f32)),
    )(x_pad, W1, degp2)

    agg1 = _msg_kernel(hs, srcw, dstw, zeros_hid).reshape(NC * N_PAD, HID)

    h2s = pl.pallas_call(
        _t2_body,
        out_shape=jax.ShapeDtypeStruct((N_PAD, HID), _f32),
    )(agg1, hs, dinvb, b1.reshape(1, HID))

    agg2 = _msg_kernel(h2s, srcw, dstw, zeros_hid).reshape(NC * N_PAD, HID)

    out = pl.pallas_call(
        _t3_body,
        out_shape=jax.ShapeDtypeStruct((N_PAD, D_OUT), _f32),
    )(agg2, h2s, dinvb, W2, b2.reshape(1, D_OUT))

    return out[:N]


# speedup vs baseline: 1.4481x; 1.4481x over previous
import functools

import jax
import jax.numpy as jnp
from jax import lax
from jax.experimental import pallas as pl
from jax.experimental.pallas import tpu as pltpu
from jax.experimental.pallas import tpu_sc as plsc

N = 10000
E = 320000
D_IN = 128
HID = 64
D_OUT = 128

NC = 2
NS = 16
NWORK = NC * NS
WIN = 128
EPW = E // NWORK
WPS = -(-EPW // WIN)
TAIL = EPW - (WPS - 1) * WIN
NBUF = 3
WMAIN = ((WPS - NBUF) // NBUF) * NBUF
N_PAD = 10240
RPS = N_PAD // NS
DEGW = 16

_mesh = plsc.VectorSubcoreMesh(core_axis_name="c", subcore_axis_name="s")
_sc_params = pltpu.CompilerParams(use_tc_tiling_on_sc=False)


@functools.partial(
    pl.kernel,
    out_type=jax.ShapeDtypeStruct((NC, N_PAD, DEGW), jnp.float32),
    mesh=_mesh,
    scratch_types=[
        pltpu.VMEM((WPS, WIN), jnp.int32),
        pltpu.VMEM((WIN, DEGW), jnp.float32),
        pltpu.VMEM((WIN, DEGW), jnp.float32),
        pltpu.VMEM_SHARED((N_PAD, DEGW), jnp.float32),
    ],
    compiler_params=_sc_params,
)
def _deg_kernel(dstw_hbm, vfull_hbm, vtail_hbm, zeros_hbm, out_hbm,
                dst_v, vfull, vtail, acc):
    c = lax.axis_index("c")
    s = lax.axis_index("s")
    wid = c * NS + s
    r0 = s * RPS
    pltpu.sync_copy(zeros_hbm.at[pl.ds(r0, RPS)], acc.at[pl.ds(r0, RPS)])
    pltpu.sync_copy(dstw_hbm.at[wid], dst_v)
    pltpu.sync_copy(vfull_hbm, vfull)
    pltpu.sync_copy(vtail_hbm, vtail)
    plsc.subcore_barrier()

    @pl.loop(0, WPS - 1)
    def _(w):
        pltpu.sync_copy(vfull, acc.at[dst_v.at[w]], add=True)

    pltpu.sync_copy(vtail, acc.at[dst_v.at[WPS - 1]], add=True)
    plsc.subcore_barrier()
    pltpu.sync_copy(acc.at[pl.ds(r0, RPS)], out_hbm.at[c, pl.ds(r0, RPS)])


@functools.partial(
    pl.kernel,
    out_type=jax.ShapeDtypeStruct((NC, N_PAD, HID), jnp.float32),
    mesh=_mesh,
    scratch_types=[
        pltpu.VMEM((WPS, WIN), jnp.int32),
        pltpu.VMEM((WPS, WIN), jnp.int32),
    ] + [pltpu.VMEM((WIN, HID), jnp.float32) for _ in range(NBUF)] + [
        pltpu.VMEM_SHARED((N_PAD, HID), jnp.float32),
    ] + [pltpu.SemaphoreType.DMA for _ in range(NBUF)],
    compiler_params=_sc_params,
)
def _msg_kernel(hs_hbm, srcw_hbm, dstw_hbm, zeros_hbm, out_hbm,
                src_v, dst_v, *rest):
    bufs = rest[:NBUF]
    acc = rest[NBUF]
    gsem = rest[NBUF + 1:]
    c = lax.axis_index("c")
    s = lax.axis_index("s")
    wid = c * NS + s
    r0 = s * RPS
    pltpu.sync_copy(zeros_hbm.at[pl.ds(r0, RPS)], acc.at[pl.ds(r0, RPS)])
    pltpu.sync_copy(srcw_hbm.at[wid], src_v)
    pltpu.sync_copy(dstw_hbm.at[wid], dst_v)
    plsc.subcore_barrier()

    def _wait_gather(i):
        pltpu.make_async_copy(hs_hbm.at[pl.ds(0, WIN)], bufs[i], gsem[i]).wait()

    def _slot(w, i, issue):
        _wait_gather(i)
        pltpu.sync_copy(bufs[i], acc.at[dst_v.at[w]], add=True)
        if issue:
            pltpu.async_copy(hs_hbm.at[src_v.at[w + NBUF]], bufs[i], gsem[i])

    for i in range(NBUF):
        pltpu.async_copy(hs_hbm.at[src_v.at[i]], bufs[i], gsem[i])

    @pl.loop(0, WMAIN, step=NBUF)
    def _(w):
        for i in range(NBUF):
            _wait_gather(i)
            pltpu.sync_copy(bufs[i], acc.at[dst_v.at[w + i]], add=True)
            pltpu.async_copy(hs_hbm.at[src_v.at[w + NBUF + i]], bufs[i],
                             gsem[i])

    for w in range(WMAIN, WPS):
        _slot(w, w % NBUF, w + NBUF < WPS)
    plsc.subcore_barrier()
    pltpu.sync_copy(acc.at[pl.ds(r0, RPS)], out_hbm.at[c, pl.ds(r0, RPS)])


def _t1_body(x_ref, w1_ref, degp_ref, hs_ref, dinvb_ref):
    h = jnp.dot(x_ref[...], w1_ref[...], preferred_element_type=jnp.float32)
    deg = degp_ref[0:N_PAD, 0:1] + degp_ref[N_PAD:2 * N_PAD, 0:1] + 1.0
    dinv = lax.rsqrt(deg)
    rows = lax.broadcasted_iota(jnp.int32, (N_PAD, 1), 0)
    dinv = jnp.where(rows < N, dinv, 0.0)
    dinvb = jnp.broadcast_to(dinv, (N_PAD, HID))
    dinvb_ref[...] = dinvb
    hs_ref[0:N, :] = h * dinvb[0:N, :]
    hs_ref[N:N_PAD, :] = jnp.zeros((N_PAD - N, HID), jnp.float32)


def _t2_body(agg_ref, hs_ref, dinvb_ref, b1_ref, h2s_ref):
    tot = agg_ref[0:N_PAD, :] + agg_ref[N_PAD:2 * N_PAD, :] + hs_ref[...]
    dinvb = dinvb_ref[...]
    out1 = dinvb * tot + b1_ref[...]
    r = jnp.maximum(out1, 0.0)
    ss = jnp.sum(r * r, axis=1, keepdims=True)
    nrm = jnp.maximum(jnp.sqrt(ss), 1e-12)
    h2s_ref[...] = (r / nrm) * dinvb


def _t3_body(agg_ref, h2s_ref, dinvb_ref, w2_ref, b2_ref, out_ref):
    pre = dinvb_ref[0:N, :] * (
        agg_ref[0:N, :] + agg_ref[N_PAD:N_PAD + N, :] + h2s_ref[0:N, :])
    out_ref[...] = (
        jnp.dot(pre, w2_ref[...], preferred_element_type=jnp.float32)
        + b2_ref[...])


_f32 = jnp.float32


def kernel(x, edge_index, W1, b1, W2, b2):
    src = edge_index[0].reshape(NWORK, EPW)
    dst = edge_index[1].reshape(NWORK, EPW)
    pad_n = WPS * WIN - EPW
    srcw = jnp.concatenate(
        [src, jnp.full((NWORK, pad_n), N, jnp.int32)], axis=1
    ).reshape(NWORK, WPS, WIN)
    dstw = jnp.concatenate(
        [dst, jnp.zeros((NWORK, pad_n), jnp.int32)], axis=1
    ).reshape(NWORK, WPS, WIN)

    zeros_deg = jnp.zeros((N_PAD, DEGW), _f32)
    zeros_hid = jnp.zeros((N_PAD, HID), _f32)
    vfull = jnp.ones((WIN, DEGW), _f32)
    vtail = jnp.zeros((WIN, DEGW), _f32).at[:TAIL].set(1.0)

    degp = _deg_kernel(dstw, vfull, vtail, zeros_deg)
    degp2 = degp.reshape(NC * N_PAD, DEGW)

    hs, dinvb = pl.pallas_call(
        _t1_body,
        out_shape=(jax.ShapeDtypeStruct((N_PAD, HID), _f32),
                   jax.ShapeDtypeStruct((N_PAD, HID), _f32)),
    )(x, W1, degp2)

    agg1 = _msg_kernel(hs, srcw, dstw, zeros_hid).reshape(NC * N_PAD, HID)

    h2s = pl.pallas_call(
        _t2_body,
        out_shape=jax.ShapeDtypeStruct((N_PAD, HID), _f32),
    )(agg1, hs, dinvb, b1.reshape(1, HID))

    agg2 = _msg_kernel(h2s, srcw, dstw, zeros_hid).reshape(NC * N_PAD, HID)

    out = pl.pallas_call(
        _t3_body,
        out_shape=jax.ShapeDtypeStruct((N, D_OUT), _f32),
    )(agg2, h2s, dinvb, W2, b2.reshape(1, D_OUT))

    return out


# speedup vs baseline: 2.3961x; 1.6546x over previous
import functools

import jax
import jax.numpy as jnp
from jax import lax
from jax.experimental import pallas as pl
from jax.experimental.pallas import tpu as pltpu
from jax.experimental.pallas import tpu_sc as plsc

N = 10000
E = 320000
D_IN = 128
HID = 64
D_OUT = 128

NC = 2
NS = 16
NWORK = NC * NS
EPW = E // NWORK
WIN = 125
WPS = EPW // WIN
NBUF = 3
WMAIN = ((WPS - NBUF) // NBUF) * NBUF
N_PAD = 10240
RPS = N_PAD // NS
DEGW = 16

_mesh = plsc.VectorSubcoreMesh(core_axis_name="c", subcore_axis_name="s")
_sc_params = pltpu.CompilerParams(use_tc_tiling_on_sc=False)


@functools.partial(
    pl.kernel,
    out_type=jax.ShapeDtypeStruct((NC, N_PAD, DEGW), jnp.float32),
    mesh=_mesh,
    scratch_types=[
        pltpu.VMEM((WPS, WIN), jnp.int32),
        pltpu.VMEM((WIN, DEGW), jnp.float32),
        pltpu.VMEM_SHARED((N_PAD, DEGW), jnp.float32),
    ],
    compiler_params=_sc_params,
)
def _deg_kernel(ei_hbm, vfull_hbm, zeros_hbm, out_hbm, dst_v, vfull, acc):
    c = lax.axis_index("c")
    s = lax.axis_index("s")
    wid = c * NS + s
    r0 = s * RPS
    pltpu.sync_copy(zeros_hbm.at[pl.ds(r0, RPS)], acc.at[pl.ds(r0, RPS)])
    pltpu.sync_copy(ei_hbm.at[1, wid], dst_v)
    pltpu.sync_copy(vfull_hbm, vfull)
    plsc.subcore_barrier()

    @pl.loop(0, WPS)
    def _(w):
        pltpu.sync_copy(vfull, acc.at[dst_v.at[w]], add=True)

    plsc.subcore_barrier()
    pltpu.sync_copy(acc.at[pl.ds(r0, RPS)], out_hbm.at[c, pl.ds(r0, RPS)])


@functools.partial(
    pl.kernel,
    out_type=jax.ShapeDtypeStruct((NC, N_PAD, HID), jnp.float32),
    mesh=_mesh,
    scratch_types=[
        pltpu.VMEM((WPS, WIN), jnp.int32),
        pltpu.VMEM((WPS, WIN), jnp.int32),
    ] + [pltpu.VMEM((WIN, HID), jnp.float32) for _ in range(NBUF)] + [
        pltpu.VMEM_SHARED((N_PAD, HID), jnp.float32),
    ] + [pltpu.SemaphoreType.DMA for _ in range(NBUF)],
    compiler_params=_sc_params,
)
def _msg_kernel(hs_hbm, ei_hbm, zeros_hbm, out_hbm, src_v, dst_v, *rest):
    bufs = rest[:NBUF]
    acc = rest[NBUF]
    gsem = rest[NBUF + 1:]
    c = lax.axis_index("c")
    s = lax.axis_index("s")
    wid = c * NS + s
    r0 = s * RPS
    pltpu.sync_copy(zeros_hbm.at[pl.ds(r0, RPS)], acc.at[pl.ds(r0, RPS)])
    pltpu.sync_copy(ei_hbm.at[0, wid], src_v)
    pltpu.sync_copy(ei_hbm.at[1, wid], dst_v)
    plsc.subcore_barrier()

    def _wait_gather(i):
        pltpu.make_async_copy(hs_hbm.at[pl.ds(0, WIN)], bufs[i], gsem[i]).wait()

    def _slot(w, i, issue):
        _wait_gather(i)
        pltpu.sync_copy(bufs[i], acc.at[dst_v.at[w]], add=True)
        if issue:
            pltpu.async_copy(hs_hbm.at[src_v.at[w + NBUF]], bufs[i], gsem[i])

    for i in range(NBUF):
        pltpu.async_copy(hs_hbm.at[src_v.at[i]], bufs[i], gsem[i])

    @pl.loop(0, WMAIN, step=NBUF)
    def _(w):
        for i in range(NBUF):
            _wait_gather(i)
            pltpu.sync_copy(bufs[i], acc.at[dst_v.at[w + i]], add=True)
            pltpu.async_copy(hs_hbm.at[src_v.at[w + NBUF + i]], bufs[i],
                             gsem[i])

    for w in range(WMAIN, WPS):
        _slot(w, w % NBUF, w + NBUF < WPS)
    plsc.subcore_barrier()
    pltpu.sync_copy(acc.at[pl.ds(r0, RPS)], out_hbm.at[c, pl.ds(r0, RPS)])


def _t1_body(x_ref, w1_ref, degp_ref, hs_ref, dinvb_ref):
    h = jnp.dot(x_ref[...], w1_ref[...], preferred_element_type=jnp.float32)
    deg = degp_ref[0:N, 0:1] + degp_ref[N_PAD:N_PAD + N, 0:1] + 1.0
    dinvb = jnp.broadcast_to(lax.rsqrt(deg), (N, HID))
    dinvb_ref[...] = dinvb
    hs_ref[...] = h * dinvb


def _t2_body(agg_ref, hs_ref, dinvb_ref, b1_ref, h2s_ref):
    tot = agg_ref[0:N, :] + agg_ref[N_PAD:N_PAD + N, :] + hs_ref[...]
    dinvb = dinvb_ref[...]
    out1 = dinvb * tot + b1_ref[...]
    r = jnp.maximum(out1, 0.0)
    ss = jnp.sum(r * r, axis=1, keepdims=True)
    nrm = jnp.maximum(jnp.sqrt(ss), 1e-12)
    h2s_ref[...] = (r / nrm) * dinvb


def _t3_body(agg_ref, h2s_ref, dinvb_ref, w2_ref, b2_ref, out_ref):
    pre = dinvb_ref[...] * (
        agg_ref[0:N, :] + agg_ref[N_PAD:N_PAD + N, :] + h2s_ref[...])
    out_ref[...] = (
        jnp.dot(pre, w2_ref[...], preferred_element_type=jnp.float32)
        + b2_ref[...])


_f32 = jnp.float32


def kernel(x, edge_index, W1, b1, W2, b2):
    ei4 = edge_index.reshape(2, NWORK, WPS, WIN)
    zeros_deg = jnp.zeros((N_PAD, DEGW), _f32)
    zeros_hid = jnp.zeros((N_PAD, HID), _f32)
    vfull = jnp.ones((WIN, DEGW), _f32)

    degp = _deg_kernel(ei4, vfull, zeros_deg)
    degp2 = degp.reshape(NC * N_PAD, DEGW)

    hs, dinvb = pl.pallas_call(
        _t1_body,
        out_shape=(jax.ShapeDtypeStruct((N, HID), _f32),
                   jax.ShapeDtypeStruct((N, HID), _f32)),
    )(x, W1, degp2)

    agg1 = _msg_kernel(hs, ei4, zeros_hid).reshape(NC * N_PAD, HID)

    h2s = pl.pallas_call(
        _t2_body,
        out_shape=jax.ShapeDtypeStruct((N, HID), _f32),
    )(agg1, hs, dinvb, b1.reshape(1, HID))

    agg2 = _msg_kernel(h2s, ei4, zeros_hid).reshape(NC * N_PAD, HID)

    out = pl.pallas_call(
        _t3_body,
        out_shape=jax.ShapeDtypeStruct((N, D_OUT), _f32),
    )(agg2, h2s, dinvb, W2, b2.reshape(1, D_OUT))

    return out


# speedup vs baseline: 2.4391x; 1.0180x over previous
import functools

import jax
import jax.numpy as jnp
from jax import lax
from jax.experimental import pallas as pl
from jax.experimental.pallas import tpu as pltpu
from jax.experimental.pallas import tpu_sc as plsc

N = 10000
E = 320000
D_IN = 128
HID = 64
D_OUT = 128

NC = 2
NS = 16
NWORK = NC * NS
EPW = E // NWORK
WIN = 125
WPS = EPW // WIN
NBUF = 4
WMAIN = ((WPS - NBUF) // NBUF) * NBUF
N_PAD = 10240
RPS = N_PAD // NS
DEGW = 16

_mesh = plsc.VectorSubcoreMesh(core_axis_name="c", subcore_axis_name="s")
_sc_params = pltpu.CompilerParams(use_tc_tiling_on_sc=False)


@functools.partial(
    pl.kernel,
    out_type=jax.ShapeDtypeStruct((NC, N_PAD, DEGW), jnp.float32),
    mesh=_mesh,
    scratch_types=[
        pltpu.VMEM((WPS, WIN), jnp.int32),
        pltpu.VMEM((WIN, DEGW), jnp.float32),
        pltpu.VMEM_SHARED((N_PAD, DEGW), jnp.float32),
    ],
    compiler_params=_sc_params,
)
def _deg_kernel(ei_hbm, vfull_hbm, zeros_hbm, out_hbm, dst_v, vfull, acc):
    c = lax.axis_index("c")
    s = lax.axis_index("s")
    wid = c * NS + s
    r0 = s * RPS
    pltpu.sync_copy(zeros_hbm.at[pl.ds(r0, RPS)], acc.at[pl.ds(r0, RPS)])
    pltpu.sync_copy(ei_hbm.at[1, wid], dst_v)
    pltpu.sync_copy(vfull_hbm, vfull)
    plsc.subcore_barrier()

    @pl.loop(0, WPS)
    def _(w):
        pltpu.sync_copy(vfull, acc.at[dst_v.at[w]], add=True)

    plsc.subcore_barrier()
    pltpu.sync_copy(acc.at[pl.ds(r0, RPS)], out_hbm.at[c, pl.ds(r0, RPS)])


@functools.partial(
    pl.kernel,
    out_type=jax.ShapeDtypeStruct((NC, N_PAD, HID), jnp.float32),
    mesh=_mesh,
    scratch_types=[
        pltpu.VMEM((WPS, WIN), jnp.int32),
        pltpu.VMEM((WPS, WIN), jnp.int32),
    ] + [pltpu.VMEM((WIN, HID), jnp.float32) for _ in range(NBUF)] + [
        pltpu.VMEM_SHARED((N_PAD, HID), jnp.float32),
    ] + [pltpu.SemaphoreType.DMA for _ in range(NBUF)],
    compiler_params=_sc_params,
)
def _msg_kernel(hs_hbm, ei_hbm, zeros_hbm, out_hbm, src_v, dst_v, *rest):
    bufs = rest[:NBUF]
    acc = rest[NBUF]
    gsem = rest[NBUF + 1:]
    c = lax.axis_index("c")
    s = lax.axis_index("s")
    wid = c * NS + s
    r0 = s * RPS
    pltpu.sync_copy(zeros_hbm.at[pl.ds(r0, RPS)], acc.at[pl.ds(r0, RPS)])
    pltpu.sync_copy(ei_hbm.at[0, wid], src_v)
    pltpu.sync_copy(ei_hbm.at[1, wid], dst_v)
    plsc.subcore_barrier()

    def _wait_gather(i):
        pltpu.make_async_copy(hs_hbm.at[pl.ds(0, WIN)], bufs[i], gsem[i]).wait()

    def _slot(w, i, issue):
        _wait_gather(i)
        pltpu.sync_copy(bufs[i], acc.at[dst_v.at[w]], add=True)
        if issue:
            pltpu.async_copy(hs_hbm.at[src_v.at[w + NBUF]], bufs[i], gsem[i])

    for i in range(NBUF):
        pltpu.async_copy(hs_hbm.at[src_v.at[i]], bufs[i], gsem[i])

    @pl.loop(0, WMAIN, step=NBUF)
    def _(w):
        for i in range(NBUF):
            _wait_gather(i)
            pltpu.sync_copy(bufs[i], acc.at[dst_v.at[w + i]], add=True)
            pltpu.async_copy(hs_hbm.at[src_v.at[w + NBUF + i]], bufs[i],
                             gsem[i])

    for w in range(WMAIN, WPS):
        _slot(w, w % NBUF, w + NBUF < WPS)
    plsc.subcore_barrier()
    pltpu.sync_copy(acc.at[pl.ds(r0, RPS)], out_hbm.at[c, pl.ds(r0, RPS)])


def _t0_body(x_ref, w1_ref, h_ref):
    h_ref[...] = jnp.dot(x_ref[...], w1_ref[...],
                         preferred_element_type=jnp.float32)


def _t1_body(h_ref, degp_ref, hs_ref, dinvb_ref):
    deg = degp_ref[0:N, 0:1] + degp_ref[N_PAD:N_PAD + N, 0:1] + 1.0
    dinvb = jnp.broadcast_to(lax.rsqrt(deg), (N, HID))
    dinvb_ref[...] = dinvb
    hs_ref[...] = h_ref[...] * dinvb


def _t2_body(agg_ref, hs_ref, dinvb_ref, b1_ref, h2s_ref):
    tot = agg_ref[0:N, :] + agg_ref[N_PAD:N_PAD + N, :] + hs_ref[...]
    dinvb = dinvb_ref[...]
    out1 = dinvb * tot + b1_ref[...]
    r = jnp.maximum(out1, 0.0)
    ss = jnp.sum(r * r, axis=1, keepdims=True)
    nrm = jnp.maximum(jnp.sqrt(ss), 1e-12)
    h2s_ref[...] = (r / nrm) * dinvb


def _t3_body(agg_ref, h2s_ref, dinvb_ref, w2_ref, b2_ref, out_ref):
    pre = dinvb_ref[...] * (
        agg_ref[0:N, :] + agg_ref[N_PAD:N_PAD + N, :] + h2s_ref[...])
    out_ref[...] = (
        jnp.dot(pre, w2_ref[...], preferred_element_type=jnp.float32)
        + b2_ref[...])


_f32 = jnp.float32


def kernel(x, edge_index, W1, b1, W2, b2):
    ei4 = edge_index.reshape(2, NWORK, WPS, WIN)
    zeros_deg = jnp.zeros((N_PAD, DEGW), _f32)
    zeros_hid = jnp.zeros((N_PAD, HID), _f32)
    vfull = jnp.ones((WIN, DEGW), _f32)

    degp = _deg_kernel(ei4, vfull, zeros_deg)
    degp2 = degp.reshape(NC * N_PAD, DEGW)
    h = pl.pallas_call(
        _t0_body, out_shape=jax.ShapeDtypeStruct((N, HID), _f32))(x, W1)

    hs, dinvb = pl.pallas_call(
        _t1_body,
        out_shape=(jax.ShapeDtypeStruct((N, HID), _f32),
                   jax.ShapeDtypeStruct((N, HID), _f32)),
    )(h, degp2)

    agg1 = _msg_kernel(hs, ei4, zeros_hid).reshape(NC * N_PAD, HID)

    h2s = pl.pallas_call(
        _t2_body,
        out_shape=jax.ShapeDtypeStruct((N, HID), _f32),
    )(agg1, hs, dinvb, b1.reshape(1, HID))

    agg2 = _msg_kernel(h2s, ei4, zeros_hid).reshape(NC * N_PAD, HID)

    out = pl.pallas_call(
        _t3_body,
        out_shape=jax.ShapeDtypeStruct((N, D_OUT), _f32),
    )(agg2, h2s, dinvb, W2, b2.reshape(1, D_OUT))

    return out
